# baseline (speedup 1.0000x reference)
import functools

import jax
import jax.numpy as jnp
from jax import lax
from jax.experimental import pallas as pl
from jax.experimental.pallas import tpu as pltpu
from jax.experimental.pallas import tpu_sc as plsc

N = 50000
E = 800000
G = 128

NC = 2
NS = 16
NW = NC * NS

NPAD = 50048
NB = 16
BLK = NPAD // NB

CH = 128
EPT = 25088
EPAD = NW * EPT
NCHUNK = EPT // CH

GPAD = 144
EPS = 1e-5

_mesh = lambda: plsc.VectorSubcoreMesh(core_axis_name="c", subcore_axis_name="s")
_sc_params = lambda: pltpu.CompilerParams(use_tc_tiling_on_sc=False)


def _sc_degree(dst_pad, ones_rows, zrows):
    @functools.partial(
        pl.kernel,
        out_type=jax.ShapeDtypeStruct((NC, NPAD, 8), jnp.float32),
        mesh=_mesh(),
        compiler_params=_sc_params(),
        scratch_types=[
            pltpu.VMEM((NCHUNK, CH), jnp.int32),
            pltpu.VMEM((CH, 8), jnp.float32),
            pltpu.VMEM_SHARED((NPAD, 8), jnp.float32),
        ],
    )
    def deg_kernel(dst_hbm, ones_hbm, zeros_hbm, out_hbm, didx, ones_v, acc):
        c = lax.axis_index("c")
        s = lax.axis_index("s")
        wid = c * NS + s
        rows_t = NPAD // NS
        r0 = s * rows_t
        pltpu.sync_copy(ones_hbm, ones_v)
        pltpu.sync_copy(zeros_hbm.at[pl.ds(r0, rows_t)], acc.at[pl.ds(r0, rows_t)])
        pltpu.sync_copy(dst_hbm.at[pl.ds(wid * NCHUNK, NCHUNK)], didx)
        plsc.subcore_barrier()

        def body(k, carry):
            pltpu.sync_copy(ones_v, acc.at[didx.at[k]], add=True)
            return carry

        lax.fori_loop(0, NCHUNK, body, 0)
        plsc.subcore_barrier()
        pltpu.sync_copy(acc.at[pl.ds(r0, rows_t)], out_hbm.at[c, pl.ds(r0, rows_t)])

    return deg_kernel(dst_pad, ones_rows, zrows)


def _sc_agg(F):
    cpb = NCHUNK if F <= 8 else NCHUNK // 4
    nblk = NCHUNK // cpb

    @functools.partial(
        pl.kernel,
        out_type=jax.ShapeDtypeStruct((NC, NPAD, F), jnp.float32),
        mesh=_mesh(),
        compiler_params=_sc_params(),
        scratch_types=[
            pltpu.VMEM((cpb, CH), jnp.int32),
            pltpu.VMEM((cpb, CH), jnp.int32),
            pltpu.VMEM((CH, F), jnp.float32),
            pltpu.VMEM((CH, F), jnp.float32),
            pltpu.VMEM_SHARED((NPAD, F), jnp.float32),
            pltpu.SemaphoreType.DMA,
            pltpu.SemaphoreType.DMA,
        ],
    )
    def agg_kernel(y_hbm, src_hbm, dst_hbm, zeros_hbm, out_hbm,
                   sidx, didx, rows0, rows1, acc, g0, g1):
        c = lax.axis_index("c")
        s = lax.axis_index("s")
        wid = c * NS + s
        rows_t = NPAD // NS
        r0 = s * rows_t
        pltpu.sync_copy(zeros_hbm.at[pl.ds(r0, rows_t)], acc.at[pl.ds(r0, rows_t)])
        plsc.subcore_barrier()

        for blk in range(nblk):
            cb = wid * NCHUNK + blk * cpb
            pltpu.sync_copy(src_hbm.at[pl.ds(cb, cpb)], sidx)
            pltpu.sync_copy(dst_hbm.at[pl.ds(cb, cpb)], didx)
            pltpu.async_copy(y_hbm.at[sidx.at[0]], rows0, g0)
            pltpu.async_copy(y_hbm.at[sidx.at[1]], rows1, g1)

            def body(kk, carry):
                for b, rows_b, sem_b in ((0, rows0, g0), (1, rows1, g1)):
                    k = kk * 2 + b
                    pltpu.make_async_copy(y_hbm.at[sidx.at[k]], rows_b, sem_b).wait()
                    pltpu.sync_copy(rows_b, acc.at[didx.at[k]], add=True)

                    @pl.when(k + 2 < cpb)
                    def _():
                        pltpu.async_copy(y_hbm.at[sidx.at[k + 2]], rows_b, sem_b)
                return carry

            lax.fori_loop(0, cpb // 2, body, 0)
            if cpb % 2:
                kt = cpb - 1
                pltpu.make_async_copy(y_hbm.at[sidx.at[kt]], rows0, g0).wait()
                pltpu.sync_copy(rows0, acc.at[didx.at[kt]], add=True)

        plsc.subcore_barrier()
        pltpu.sync_copy(acc.at[pl.ds(r0, rows_t)], out_hbm.at[c, pl.ds(r0, rows_t)])

    return agg_kernel


def _sc_pool(h_pad, batch_pad, neginf_rows):
    nchunks_total = NPAD // CH

    @functools.partial(
        pl.kernel,
        out_type=jax.ShapeDtypeStruct((NC, GPAD, 128), jnp.float32),
        mesh=_mesh(),
        compiler_params=_sc_params(),
        scratch_types=[
            pltpu.VMEM((CH,), jnp.int32),
            pltpu.VMEM((CH, 128), jnp.float32),
            pltpu.VMEM((GPAD, 128), jnp.float32),
            pltpu.VMEM((2, 9, 128), jnp.float32),
            pltpu.VMEM_SHARED((NS, GPAD, 128), jnp.float32),
        ],
    )
    def pool_kernel(h_hbm, b_hbm, neg_hbm, out_hbm, bidx, hrows, acc, rbuf, shared):
        c = lax.axis_index("c")
        s = lax.axis_index("s")
        wid = c * NS + s
        pltpu.sync_copy(neg_hbm, acc)
        nch = (nchunks_total - wid + NW - 1) // NW

        def chunk_body(k, carry):
            base = (wid + k * NW) * CH
            pltpu.sync_copy(b_hbm.at[pl.ds(base, CH)], bidx)
            pltpu.sync_copy(h_hbm.at[pl.ds(base, CH)], hrows)

            def group_body(q, carry2):
                bvec = bidx[pl.ds(q * 16, 16)]
                for r in range(16):
                    bi = bvec[r]
                    rr = q * 16 + r
                    for j in range(8):
                        v = hrows[rr, pl.ds(j * 16, 16)]
                        a = acc[bi, pl.ds(j * 16, 16)]
                        acc[bi, pl.ds(j * 16, 16)] = jnp.maximum(a, v)
                return carry2

            lax.fori_loop(0, CH // 16, group_body, 0)
            return carry

        lax.fori_loop(0, nch, chunk_body, 0)
        pltpu.sync_copy(acc, shared.at[s])
        plsc.subcore_barrier()
        g0 = s * 9
        pltpu.sync_copy(shared.at[0, pl.ds(g0, 9)], rbuf.at[0])

        def red_body(i, carry):
            pltpu.sync_copy(shared.at[i, pl.ds(g0, 9)], rbuf.at[1])
            for r in range(9):
                for j in range(8):
                    a = rbuf[0, r, pl.ds(j * 16, 16)]
                    v = rbuf[1, r, pl.ds(j * 16, 16)]
                    rbuf[0, r, pl.ds(j * 16, 16)] = jnp.maximum(a, v)
            return carry

        lax.fori_loop(1, NS, red_body, 0)
        pltpu.sync_copy(rbuf.at[0], out_hbm.at[c, pl.ds(g0, 9)])

    return pool_kernel(h_pad, batch_pad, neginf_rows)


def _tc_prep(deg_p, x_pad):
    def body(dp_ref, x_ref, dinv_ref, y1_ref):
        d = 1.0 + dp_ref[0, :, 0:1] + dp_ref[1, :, 0:1]
        dinv = lax.rsqrt(d)
        dinv_ref[...] = dinv
        y1_ref[...] = dinv * x_ref[...]

    return pl.pallas_call(
        body,
        grid=(NB,),
        in_specs=[
            pl.BlockSpec((NC, BLK, 8), lambda i: (0, i, 0)),
            pl.BlockSpec((BLK, 8), lambda i: (i, 0)),
        ],
        out_specs=[
            pl.BlockSpec((BLK, 1), lambda i: (i, 0)),
            pl.BlockSpec((BLK, 8), lambda i: (i, 0)),
        ],
        out_shape=[
            jax.ShapeDtypeStruct((NPAD, 1), jnp.float32),
            jax.ShapeDtypeStruct((NPAD, 8), jnp.float32),
        ],
    )(deg_p, x_pad)


def _tc_conv(z_p, y, dinv, W, b2d):
    Fi, Fo = W.shape

    def body(z_ref, y_ref, dinv_ref, w_ref, b_ref, t_ref, s_ref):
        i = pl.program_id(0)
        u = dinv_ref[...] * (z_ref[0] + z_ref[1] + y_ref[...])
        t = jnp.dot(u, w_ref[...], preferred_element_type=jnp.float32) + b_ref[...]
        t = jnp.maximum(t, 0.0)
        t_ref[...] = t
        rows = i * BLK + lax.broadcasted_iota(jnp.int32, (BLK, 1), 0)
        tm = jnp.where(rows < N, t, 0.0)

        @pl.when(i == 0)
        def _():
            s_ref[...] = jnp.zeros_like(s_ref)

        s_ref[0:1, :] += jnp.sum(tm, axis=0, keepdims=True)

    return pl.pallas_call(
        body,
        grid=(NB,),
        in_specs=[
            pl.BlockSpec((NC, BLK, Fi), lambda i: (0, i, 0)),
            pl.BlockSpec((BLK, Fi), lambda i: (i, 0)),
            pl.BlockSpec((BLK, 1), lambda i: (i, 0)),
            pl.BlockSpec((Fi, Fo), lambda i: (0, 0)),
            pl.BlockSpec((1, Fo), lambda i: (0, 0)),
        ],
        out_specs=[
            pl.BlockSpec((BLK, Fo), lambda i: (i, 0)),
            pl.BlockSpec((8, Fo), lambda i: (0, 0)),
        ],
        out_shape=[
            jax.ShapeDtypeStruct((NPAD, Fo), jnp.float32),
            jax.ShapeDtypeStruct((8, Fo), jnp.float32),
        ],
    )(z_p, y, dinv, W, b2d)


def _tc_bn(t, sums, dinv, g2d, bt2d, scale_by_dinv):
    Fo = t.shape[1]

    def body(t_ref, s_ref, dinv_ref, g_ref, bt_ref, o_ref, v_ref):
        ph = pl.program_id(0)
        i = pl.program_id(1)
        m = s_ref[0:1, :] * (1.0 / N)

        @pl.when(ph == 0)
        def _():
            rows = i * BLK + lax.broadcasted_iota(jnp.int32, (BLK, 1), 0)
            d = jnp.where(rows < N, t_ref[...] - m, 0.0)

            @pl.when(i == 0)
            def _():
                v_ref[...] = jnp.zeros_like(v_ref)

            v_ref[0:1, :] += jnp.sum(d * d, axis=0, keepdims=True)

        @pl.when(ph == 1)
        def _():
            var = v_ref[0:1, :] * (1.0 / N)
            a = g_ref[...] * lax.rsqrt(var + EPS)
            h = a * (t_ref[...] - m) + bt_ref[...]
            if scale_by_dinv:
                h = dinv_ref[...] * h
            o_ref[...] = h

    return pl.pallas_call(
        body,
        grid=(2, NB),
        in_specs=[
            pl.BlockSpec((BLK, Fo), lambda p, i: (i, 0)),
            pl.BlockSpec((8, Fo), lambda p, i: (0, 0)),
            pl.BlockSpec((BLK, 1), lambda p, i: (i, 0)),
            pl.BlockSpec((1, Fo), lambda p, i: (0, 0)),
            pl.BlockSpec((1, Fo), lambda p, i: (0, 0)),
        ],
        out_specs=pl.BlockSpec((BLK, Fo), lambda p, i: (i, 0)),
        out_shape=jax.ShapeDtypeStruct((NPAD, Fo), jnp.float32),
        scratch_shapes=[pltpu.VMEM((8, Fo), jnp.float32)],
    )(t, sums, dinv, g2d, bt2d)


def _tc_head(pool_p, Wl, bl2d):
    def body(p_ref, w_ref, b_ref, o_ref):
        p = jnp.maximum(p_ref[0], p_ref[1])[:G]
        v = jnp.dot(p, w_ref[...], preferred_element_type=jnp.float32) + b_ref[...]
        mx = jnp.max(v, axis=1, keepdims=True)
        e = jnp.exp(v - mx)
        o_ref[...] = (v - mx) - jnp.log(jnp.sum(e, axis=1, keepdims=True))

    return pl.pallas_call(
        body,
        out_shape=jax.ShapeDtypeStruct((G, 3), jnp.float32),
    )(pool_p, Wl, bl2d)


def kernel(x, edge_index, batch_index, W1, b1, g1, bt1, W2, b2, g2, bt2,
           W3, b3, g3, bt3, Wl, bl):
    i32 = jnp.int32
    src_pad = jnp.concatenate([edge_index[0].astype(i32),
                               jnp.zeros((EPAD - E,), i32)]).reshape(EPAD // CH, CH)
    dst_pad = jnp.concatenate([edge_index[1].astype(i32),
                               jnp.full((EPAD - E,), N, i32)]).reshape(EPAD // CH, CH)
    batch_pad = jnp.concatenate([batch_index.astype(i32),
                                 jnp.full((NPAD - N,), G, i32)])
    x_pad = jnp.zeros((NPAD, 8), jnp.float32).at[:N, :2].set(x)

    ones_rows = jnp.ones((CH, 8), jnp.float32)
    zrows1 = jnp.zeros((NPAD, 8), jnp.float32)
    neginf_rows = jnp.full((GPAD, 128), -jnp.inf, jnp.float32)

    deg_p = _sc_degree(dst_pad, ones_rows, zrows1)
    dinv = lax.rsqrt(1.0 + deg_p[0, :, 0:1] + deg_p[1, :, 0:1])
    y = dinv * x_pad

    W1p = jnp.zeros((8, 8), jnp.float32).at[:2].set(W1)
    layers = [(W1p, b1, g1, bt1), (W2, b2, g2, bt2), (W3, b3, g3, bt3)]
    for li, (W, b, g, bt) in enumerate(layers):
        Fi = W.shape[0]
        z_p = _sc_agg(Fi)(y, src_pad, dst_pad, jnp.zeros((NPAD, Fi), jnp.float32))
        u = dinv * (z_p[0] + z_p[1] + y)
        t = jnp.maximum(u @ W + b, 0.0)
        tm = jnp.where(jnp.arange(NPAD)[:, None] < N, t, 0.0)
        m = jnp.sum(tm, 0) / N
        v = jnp.sum(jnp.where(jnp.arange(NPAD)[:, None] < N, (t - m) ** 2, 0.0), 0) / N
        h = g * (t - m) * lax.rsqrt(v + EPS) + bt
        y = dinv * h if li < 2 else h

    pool_p = _sc_pool(y, batch_pad, neginf_rows)
    p = jnp.maximum(pool_p[0], pool_p[1])[:G]
    o = p @ Wl + bl
    return jax.nn.log_softmax(o, axis=1)


# speedup vs baseline: 1.0628x; 1.0628x over previous
import functools

import jax
import jax.numpy as jnp
from jax import lax
from jax.experimental import pallas as pl
from jax.experimental.pallas import tpu as pltpu
from jax.experimental.pallas import tpu_sc as plsc

N = 50000
E = 800000
G = 128

NC = 2
NS = 16
NW = NC * NS

NPAD = 50048
NB = 16
BLK = NPAD // NB

CH = 128
EPT = 25088
EPAD = NW * EPT
NCHUNK = EPT // CH

GPAD = 144
EPS = 1e-5

_mesh = lambda: plsc.VectorSubcoreMesh(core_axis_name="c", subcore_axis_name="s")
_sc_params = lambda: pltpu.CompilerParams(use_tc_tiling_on_sc=False)


def _sc_degree(dst_pad, ones_rows, zrows):
    @functools.partial(
        pl.kernel,
        out_type=jax.ShapeDtypeStruct((NC, NPAD, 8), jnp.float32),
        mesh=_mesh(),
        compiler_params=_sc_params(),
        scratch_types=[
            pltpu.VMEM((NCHUNK, CH), jnp.int32),
            pltpu.VMEM((CH, 8), jnp.float32),
            pltpu.VMEM_SHARED((NPAD, 8), jnp.float32),
        ],
    )
    def deg_kernel(dst_hbm, ones_hbm, zeros_hbm, out_hbm, didx, ones_v, acc):
        c = lax.axis_index("c")
        s = lax.axis_index("s")
        wid = c * NS + s
        rows_t = NPAD // NS
        r0 = s * rows_t
        pltpu.sync_copy(ones_hbm, ones_v)
        pltpu.sync_copy(zeros_hbm.at[pl.ds(r0, rows_t)], acc.at[pl.ds(r0, rows_t)])
        pltpu.sync_copy(dst_hbm.at[pl.ds(wid * NCHUNK, NCHUNK)], didx)
        plsc.subcore_barrier()

        def body(k, carry):
            pltpu.sync_copy(ones_v, acc.at[didx.at[k]], add=True)
            return carry

        lax.fori_loop(0, NCHUNK, body, 0)
        plsc.subcore_barrier()
        pltpu.sync_copy(acc.at[pl.ds(r0, rows_t)], out_hbm.at[c, pl.ds(r0, rows_t)])

    return deg_kernel(dst_pad, ones_rows, zrows)


def _sc_agg(F):
    cpb = NCHUNK if F <= 8 else NCHUNK // 4
    nblk = NCHUNK // cpb

    @functools.partial(
        pl.kernel,
        out_type=jax.ShapeDtypeStruct((NC, NPAD, F), jnp.float32),
        mesh=_mesh(),
        compiler_params=_sc_params(),
        scratch_types=[
            pltpu.VMEM((cpb, CH), jnp.int32),
            pltpu.VMEM((cpb, CH), jnp.int32),
            pltpu.VMEM((4, CH, F), jnp.float32),
            pltpu.VMEM_SHARED((NPAD, F), jnp.float32),
            [pltpu.SemaphoreType.DMA] * 4,
            [pltpu.SemaphoreType.DMA] * 4,
        ],
    )
    def agg_kernel(y_hbm, src_hbm, dst_hbm, zeros_hbm, out_hbm,
                   sidx, didx, rows, acc, gsems, ssems):
        c = lax.axis_index("c")
        s = lax.axis_index("s")
        wid = c * NS + s
        rows_t = NPAD // NS
        r0 = s * rows_t
        pltpu.sync_copy(zeros_hbm.at[pl.ds(r0, rows_t)], acc.at[pl.ds(r0, rows_t)])
        plsc.subcore_barrier()

        for blk in range(nblk):
            cb = wid * NCHUNK + blk * cpb
            pltpu.sync_copy(src_hbm.at[pl.ds(cb, cpb)], sidx)
            pltpu.sync_copy(dst_hbm.at[pl.ds(cb, cpb)], didx)
            for b in range(2):
                pltpu.async_copy(y_hbm.at[sidx.at[b]], rows.at[b], gsems[b])

            def step(k, b, guard):
                b2 = (b + 2) % 4
                pltpu.make_async_copy(y_hbm.at[sidx.at[k]], rows.at[b],
                                      gsems[b]).wait()
                pltpu.async_copy(rows.at[b], acc.at[didx.at[k]], ssems[b],
                                 add=True)
                if guard:
                    @pl.when(k >= 2)
                    def _():
                        pltpu.make_async_copy(rows.at[b2], acc.at[didx.at[0]],
                                              ssems[b2]).wait()
                    @pl.when(k + 2 < cpb)
                    def _():
                        pltpu.async_copy(y_hbm.at[sidx.at[k + 2]], rows.at[b2],
                                         gsems[b2])

            def body(kk, carry):
                for j in range(4):
                    step(kk * 4 + j, j, True)
                return carry

            lax.fori_loop(0, cpb // 4, body, 0)
            for k in range((cpb // 4) * 4, cpb):
                b, b2 = k % 4, (k + 2) % 4
                pltpu.make_async_copy(y_hbm.at[sidx.at[k]], rows.at[b],
                                      gsems[b]).wait()
                pltpu.async_copy(rows.at[b], acc.at[didx.at[k]], ssems[b],
                                 add=True)
                pltpu.make_async_copy(rows.at[b2], acc.at[didx.at[0]],
                                      ssems[b2]).wait()
                if k + 2 < cpb:
                    pltpu.async_copy(y_hbm.at[sidx.at[k + 2]], rows.at[b2],
                                     gsems[b2])
            for k in (cpb - 2, cpb - 1):
                pltpu.make_async_copy(rows.at[k % 4], acc.at[didx.at[0]],
                                      ssems[k % 4]).wait()

        plsc.subcore_barrier()
        pltpu.sync_copy(acc.at[pl.ds(r0, rows_t)], out_hbm.at[c, pl.ds(r0, rows_t)])

    return agg_kernel


def _sc_pool(h_pad, batch_pad, neginf_rows):
    nchunks_total = NPAD // CH

    @functools.partial(
        pl.kernel,
        out_type=jax.ShapeDtypeStruct((NC, GPAD, 128), jnp.float32),
        mesh=_mesh(),
        compiler_params=_sc_params(),
        scratch_types=[
            pltpu.VMEM((CH,), jnp.int32),
            pltpu.VMEM((CH, 128), jnp.float32),
            pltpu.VMEM((GPAD, 128), jnp.float32),
            pltpu.VMEM((2, 9, 128), jnp.float32),
            pltpu.VMEM_SHARED((NS, GPAD, 128), jnp.float32),
        ],
    )
    def pool_kernel(h_hbm, b_hbm, neg_hbm, out_hbm, bidx, hrows, acc, rbuf, shared):
        c = lax.axis_index("c")
        s = lax.axis_index("s")
        wid = c * NS + s
        pltpu.sync_copy(neg_hbm, acc)
        nch = (nchunks_total - wid + NW - 1) // NW

        def chunk_body(k, carry):
            base = (wid + k * NW) * CH
            pltpu.sync_copy(b_hbm.at[pl.ds(base, CH)], bidx)
            pltpu.sync_copy(h_hbm.at[pl.ds(base, CH)], hrows)

            def group_body(q, carry2):
                bvec = bidx[pl.ds(q * 16, 16)]
                for r in range(16):
                    bi = bvec[r]
                    rr = q * 16 + r
                    for j in range(8):
                        v = hrows[rr, pl.ds(j * 16, 16)]
                        a = acc[bi, pl.ds(j * 16, 16)]
                        acc[bi, pl.ds(j * 16, 16)] = jnp.maximum(a, v)
                return carry2

            lax.fori_loop(0, CH // 16, group_body, 0)
            return carry

        lax.fori_loop(0, nch, chunk_body, 0)
        pltpu.sync_copy(acc, shared.at[s])
        plsc.subcore_barrier()
        g0 = s * 9
        pltpu.sync_copy(shared.at[0, pl.ds(g0, 9)], rbuf.at[0])

        def red_body(i, carry):
            pltpu.sync_copy(shared.at[i, pl.ds(g0, 9)], rbuf.at[1])
            for r in range(9):
                for j in range(8):
                    a = rbuf[0, r, pl.ds(j * 16, 16)]
                    v = rbuf[1, r, pl.ds(j * 16, 16)]
                    rbuf[0, r, pl.ds(j * 16, 16)] = jnp.maximum(a, v)
            return carry

        lax.fori_loop(1, NS, red_body, 0)
        pltpu.sync_copy(rbuf.at[0], out_hbm.at[c, pl.ds(g0, 9)])

    return pool_kernel(h_pad, batch_pad, neginf_rows)


def _tc_prep(deg_p, x_pad):
    def body(dp_ref, x_ref, dinv_ref, y1_ref):
        d = 1.0 + dp_ref[0, :, 0:1] + dp_ref[1, :, 0:1]
        dinv = lax.rsqrt(d)
        dinv_ref[...] = dinv
        y1_ref[...] = dinv * x_ref[...]

    return pl.pallas_call(
        body,
        grid=(NB,),
        in_specs=[
            pl.BlockSpec((NC, BLK, 8), lambda i: (0, i, 0)),
            pl.BlockSpec((BLK, 8), lambda i: (i, 0)),
        ],
        out_specs=[
            pl.BlockSpec((BLK, 1), lambda i: (i, 0)),
            pl.BlockSpec((BLK, 8), lambda i: (i, 0)),
        ],
        out_shape=[
            jax.ShapeDtypeStruct((NPAD, 1), jnp.float32),
            jax.ShapeDtypeStruct((NPAD, 8), jnp.float32),
        ],
    )(deg_p, x_pad)


def _tc_conv(z_p, y, dinv, W, b2d):
    Fi, Fo = W.shape

    def body(z_ref, y_ref, dinv_ref, w_ref, b_ref, t_ref, s_ref):
        i = pl.program_id(0)
        u = dinv_ref[...] * (z_ref[0] + z_ref[1] + y_ref[...])
        t = jnp.dot(u, w_ref[...], preferred_element_type=jnp.float32) + b_ref[...]
        t = jnp.maximum(t, 0.0)
        t_ref[...] = t
        rows = i * BLK + lax.broadcasted_iota(jnp.int32, (BLK, 1), 0)
        tm = jnp.where(rows < N, t, 0.0)

        @pl.when(i == 0)
        def _():
            s_ref[...] = jnp.zeros_like(s_ref)

        s_ref[0:1, :] += jnp.sum(tm, axis=0, keepdims=True)

    return pl.pallas_call(
        body,
        grid=(NB,),
        in_specs=[
            pl.BlockSpec((NC, BLK, Fi), lambda i: (0, i, 0)),
            pl.BlockSpec((BLK, Fi), lambda i: (i, 0)),
            pl.BlockSpec((BLK, 1), lambda i: (i, 0)),
            pl.BlockSpec((Fi, Fo), lambda i: (0, 0)),
            pl.BlockSpec((1, Fo), lambda i: (0, 0)),
        ],
        out_specs=[
            pl.BlockSpec((BLK, Fo), lambda i: (i, 0)),
            pl.BlockSpec((8, Fo), lambda i: (0, 0)),
        ],
        out_shape=[
            jax.ShapeDtypeStruct((NPAD, Fo), jnp.float32),
            jax.ShapeDtypeStruct((8, Fo), jnp.float32),
        ],
    )(z_p, y, dinv, W, b2d)


def _tc_bn(t, sums, dinv, g2d, bt2d, scale_by_dinv):
    Fo = t.shape[1]

    def body(t_ref, s_ref, dinv_ref, g_ref, bt_ref, o_ref, v_ref):
        ph = pl.program_id(0)
        i = pl.program_id(1)
        m = s_ref[0:1, :] * (1.0 / N)

        @pl.when(ph == 0)
        def _():
            rows = i * BLK + lax.broadcasted_iota(jnp.int32, (BLK, 1), 0)
            d = jnp.where(rows < N, t_ref[...] - m, 0.0)

            @pl.when(i == 0)
            def _():
                v_ref[...] = jnp.zeros_like(v_ref)

            v_ref[0:1, :] += jnp.sum(d * d, axis=0, keepdims=True)

        @pl.when(ph == 1)
        def _():
            var = v_ref[0:1, :] * (1.0 / N)
            a = g_ref[...] * lax.rsqrt(var + EPS)
            h = a * (t_ref[...] - m) + bt_ref[...]
            if scale_by_dinv:
                h = dinv_ref[...] * h
            o_ref[...] = h

    return pl.pallas_call(
        body,
        grid=(2, NB),
        in_specs=[
            pl.BlockSpec((BLK, Fo), lambda p, i: (i, 0)),
            pl.BlockSpec((8, Fo), lambda p, i: (0, 0)),
            pl.BlockSpec((BLK, 1), lambda p, i: (i, 0)),
            pl.BlockSpec((1, Fo), lambda p, i: (0, 0)),
            pl.BlockSpec((1, Fo), lambda p, i: (0, 0)),
        ],
        out_specs=pl.BlockSpec((BLK, Fo), lambda p, i: (i, 0)),
        out_shape=jax.ShapeDtypeStruct((NPAD, Fo), jnp.float32),
        scratch_shapes=[pltpu.VMEM((8, Fo), jnp.float32)],
    )(t, sums, dinv, g2d, bt2d)


def _tc_head(pool_p, Wl, bl2d):
    def body(p_ref, w_ref, b_ref, o_ref):
        p = jnp.maximum(p_ref[0], p_ref[1])[:G]
        v = jnp.dot(p, w_ref[...], preferred_element_type=jnp.float32) + b_ref[...]
        mx = jnp.max(v, axis=1, keepdims=True)
        e = jnp.exp(v - mx)
        o_ref[...] = (v - mx) - jnp.log(jnp.sum(e, axis=1, keepdims=True))

    return pl.pallas_call(
        body,
        out_shape=jax.ShapeDtypeStruct((G, 3), jnp.float32),
    )(pool_p, Wl, bl2d)


def kernel(x, edge_index, batch_index, W1, b1, g1, bt1, W2, b2, g2, bt2,
           W3, b3, g3, bt3, Wl, bl):
    i32 = jnp.int32
    src_pad = jnp.concatenate([edge_index[0].astype(i32),
                               jnp.zeros((EPAD - E,), i32)]).reshape(EPAD // CH, CH)
    dst_pad = jnp.concatenate([edge_index[1].astype(i32),
                               jnp.full((EPAD - E,), N, i32)]).reshape(EPAD // CH, CH)
    batch_pad = jnp.concatenate([batch_index.astype(i32),
                                 jnp.full((NPAD - N,), G, i32)])
    x_pad = jnp.zeros((NPAD, 8), jnp.float32).at[:N, :2].set(x)

    ones_rows = jnp.ones((CH, 8), jnp.float32)
    zrows1 = jnp.zeros((NPAD, 8), jnp.float32)
    neginf_rows = jnp.full((GPAD, 128), -jnp.inf, jnp.float32)

    deg_p = _sc_degree(dst_pad, ones_rows, zrows1)
    dinv, y = _tc_prep(deg_p, x_pad)

    W1p = jnp.zeros((8, 8), jnp.float32).at[:2].set(W1)
    layers = [(W1p, b1, g1, bt1), (W2, b2, g2, bt2), (W3, b3, g3, bt3)]
    for li, (W, b, g, bt) in enumerate(layers):
        Fi = W.shape[0]
        z_p = _sc_agg(Fi)(y, src_pad, dst_pad, jnp.zeros((NPAD, Fi), jnp.float32))
        t, sums = _tc_conv(z_p, y, dinv, W, b.reshape(1, -1))
        y = _tc_bn(t, sums, dinv, g.reshape(1, -1), bt.reshape(1, -1),
                   scale_by_dinv=(li < 2))

    pool_p = _sc_pool(y, batch_pad, neginf_rows)
    return _tc_head(pool_p, Wl, bl.reshape(1, -1))


# speedup vs baseline: 1.1252x; 1.0586x over previous
import functools

import jax
import jax.numpy as jnp
from jax import lax
from jax.experimental import pallas as pl
from jax.experimental.pallas import tpu as pltpu
from jax.experimental.pallas import tpu_sc as plsc

N = 50000
E = 800000
G = 128

NC = 2
NS = 16
NW = NC * NS

NPAD = 50048
NB = 16
BLK = NPAD // NB

CH = 128
EPT = 25088
EPAD = NW * EPT
NCHUNK = EPT // CH

GPAD = 144
EPS = 1e-5

_mesh = lambda: plsc.VectorSubcoreMesh(core_axis_name="c", subcore_axis_name="s")
_sc_params = lambda: pltpu.CompilerParams(use_tc_tiling_on_sc=False)


def _sc_degree(dst_pad, ones_rows, zrows):
    @functools.partial(
        pl.kernel,
        out_type=jax.ShapeDtypeStruct((NC, NPAD, 8), jnp.float32),
        mesh=_mesh(),
        compiler_params=_sc_params(),
        scratch_types=[
            pltpu.VMEM((NCHUNK, CH), jnp.int32),
            pltpu.VMEM((CH, 8), jnp.float32),
            pltpu.VMEM_SHARED((NPAD, 8), jnp.float32),
        ],
    )
    def deg_kernel(dst_hbm, ones_hbm, zeros_hbm, out_hbm, didx, ones_v, acc):
        c = lax.axis_index("c")
        s = lax.axis_index("s")
        wid = c * NS + s
        rows_t = NPAD // NS
        r0 = s * rows_t
        pltpu.sync_copy(ones_hbm, ones_v)
        pltpu.sync_copy(zeros_hbm.at[pl.ds(r0, rows_t)], acc.at[pl.ds(r0, rows_t)])
        pltpu.sync_copy(dst_hbm.at[pl.ds(wid * NCHUNK, NCHUNK)], didx)
        plsc.subcore_barrier()

        def body(k, carry):
            pltpu.sync_copy(ones_v, acc.at[didx.at[k]], add=True)
            return carry

        lax.fori_loop(0, NCHUNK, body, 0)
        plsc.subcore_barrier()
        pltpu.sync_copy(acc.at[pl.ds(r0, rows_t)], out_hbm.at[c, pl.ds(r0, rows_t)])

    return deg_kernel(dst_pad, ones_rows, zrows)


def _sc_agg(F):
    cpb = NCHUNK if F <= 8 else NCHUNK // 4
    nblk = NCHUNK // cpb

    @functools.partial(
        pl.kernel,
        out_type=jax.ShapeDtypeStruct((NC, NPAD, F), jnp.float32),
        mesh=_mesh(),
        compiler_params=_sc_params(),
        scratch_types=[
            pltpu.VMEM((cpb, CH), jnp.int32),
            pltpu.VMEM((cpb, CH), jnp.int32),
            pltpu.VMEM((4, CH, F), jnp.float32),
            pltpu.VMEM_SHARED((NPAD, F), jnp.float32),
            [pltpu.SemaphoreType.DMA] * 4,
            [pltpu.SemaphoreType.DMA] * 4,
        ],
    )
    def agg_kernel(y_hbm, src_hbm, dst_hbm, zeros_hbm, out_hbm,
                   sidx, didx, rows, acc, gsems, ssems):
        c = lax.axis_index("c")
        s = lax.axis_index("s")
        wid = c * NS + s
        rows_t = NPAD // NS
        r0 = s * rows_t
        pltpu.sync_copy(zeros_hbm.at[pl.ds(r0, rows_t)], acc.at[pl.ds(r0, rows_t)])
        plsc.subcore_barrier()

        for blk in range(nblk):
            cb = wid * NCHUNK + blk * cpb
            pltpu.sync_copy(src_hbm.at[pl.ds(cb, cpb)], sidx)
            pltpu.sync_copy(dst_hbm.at[pl.ds(cb, cpb)], didx)
            for b in range(2):
                pltpu.async_copy(y_hbm.at[sidx.at[b]], rows.at[b], gsems[b])

            def step(k, b, guard):
                b2 = (b + 2) % 4
                pltpu.make_async_copy(y_hbm.at[sidx.at[k]], rows.at[b],
                                      gsems[b]).wait()
                pltpu.async_copy(rows.at[b], acc.at[didx.at[k]], ssems[b],
                                 add=True)
                if guard:
                    @pl.when(k >= 2)
                    def _():
                        pltpu.make_async_copy(rows.at[b2], acc.at[didx.at[0]],
                                              ssems[b2]).wait()
                    @pl.when(k + 2 < cpb)
                    def _():
                        pltpu.async_copy(y_hbm.at[sidx.at[k + 2]], rows.at[b2],
                                         gsems[b2])

            def body(kk, carry):
                for j in range(4):
                    step(kk * 4 + j, j, True)
                return carry

            lax.fori_loop(0, cpb // 4, body, 0)
            for k in range((cpb // 4) * 4, cpb):
                b, b2 = k % 4, (k + 2) % 4
                pltpu.make_async_copy(y_hbm.at[sidx.at[k]], rows.at[b],
                                      gsems[b]).wait()
                pltpu.async_copy(rows.at[b], acc.at[didx.at[k]], ssems[b],
                                 add=True)
                pltpu.make_async_copy(rows.at[b2], acc.at[didx.at[0]],
                                      ssems[b2]).wait()
                if k + 2 < cpb:
                    pltpu.async_copy(y_hbm.at[sidx.at[k + 2]], rows.at[b2],
                                     gsems[b2])
            for k in (cpb - 2, cpb - 1):
                pltpu.make_async_copy(rows.at[k % 4], acc.at[didx.at[0]],
                                      ssems[k % 4]).wait()

        plsc.subcore_barrier()
        pltpu.sync_copy(acc.at[pl.ds(r0, rows_t)], out_hbm.at[c, pl.ds(r0, rows_t)])

    return agg_kernel


def _sc_pool(h_pad, batch_pad, neginf_rows):
    nchunks_total = NPAD // CH

    @functools.partial(
        pl.kernel,
        out_type=jax.ShapeDtypeStruct((NC, GPAD, 128), jnp.float32),
        mesh=_mesh(),
        compiler_params=_sc_params(),
        scratch_types=[
            pltpu.VMEM((CH,), jnp.int32),
            pltpu.VMEM((CH, 128), jnp.float32),
            pltpu.VMEM((GPAD, 128), jnp.float32),
            pltpu.VMEM((2, 9, 128), jnp.float32),
            pltpu.VMEM_SHARED((NS, GPAD, 128), jnp.float32),
        ],
    )
    def pool_kernel(h_hbm, b_hbm, neg_hbm, out_hbm, bidx, hrows, acc, rbuf, shared):
        c = lax.axis_index("c")
        s = lax.axis_index("s")
        wid = c * NS + s
        pltpu.sync_copy(neg_hbm, acc)
        nch = (nchunks_total - wid + NW - 1) // NW

        def chunk_body(k, carry):
            base = (wid + k * NW) * CH
            pltpu.sync_copy(b_hbm.at[pl.ds(base, CH)], bidx)
            pltpu.sync_copy(h_hbm.at[pl.ds(base, CH)], hrows)

            def group_body(q, carry2):
                bvec = bidx[pl.ds(q * 16, 16)]
                for r in range(16):
                    bi = bvec[r]
                    rr = q * 16 + r
                    for j in range(8):
                        v = hrows[rr, pl.ds(j * 16, 16)]
                        a = acc[bi, pl.ds(j * 16, 16)]
                        acc[bi, pl.ds(j * 16, 16)] = jnp.maximum(a, v)
                return carry2

            lax.fori_loop(0, CH // 16, group_body, 0)
            return carry

        lax.fori_loop(0, nch, chunk_body, 0)
        pltpu.sync_copy(acc, shared.at[s])
        plsc.subcore_barrier()
        g0 = s * 9
        pltpu.sync_copy(shared.at[0, pl.ds(g0, 9)], rbuf.at[0])

        def red_body(i, carry):
            pltpu.sync_copy(shared.at[i, pl.ds(g0, 9)], rbuf.at[1])
            for r in range(9):
                for j in range(8):
                    a = rbuf[0, r, pl.ds(j * 16, 16)]
                    v = rbuf[1, r, pl.ds(j * 16, 16)]
                    rbuf[0, r, pl.ds(j * 16, 16)] = jnp.maximum(a, v)
            return carry

        lax.fori_loop(1, NS, red_body, 0)
        pltpu.sync_copy(rbuf.at[0], out_hbm.at[c, pl.ds(g0, 9)])

    return pool_kernel(h_pad, batch_pad, neginf_rows)


def _tc_prep(deg_p, x_pad):
    def body(dp_ref, x_ref, dinv_ref, y1_ref):
        d = 1.0 + dp_ref[0, :, 0:1] + dp_ref[1, :, 0:1]
        dinv = lax.rsqrt(d)
        dinv_ref[...] = dinv
        y1_ref[...] = dinv * x_ref[...]

    return pl.pallas_call(
        body,
        grid=(NB,),
        in_specs=[
            pl.BlockSpec((NC, BLK, 8), lambda i: (0, i, 0)),
            pl.BlockSpec((BLK, 8), lambda i: (i, 0)),
        ],
        out_specs=[
            pl.BlockSpec((BLK, 1), lambda i: (i, 0)),
            pl.BlockSpec((BLK, 8), lambda i: (i, 0)),
        ],
        out_shape=[
            jax.ShapeDtypeStruct((NPAD, 1), jnp.float32),
            jax.ShapeDtypeStruct((NPAD, 8), jnp.float32),
        ],
    )(deg_p, x_pad)


def _tc_conv(z_p, y, dinv, W, b2d):
    Fi, Fo = W.shape

    def body(z_ref, y_ref, dinv_ref, w_ref, b_ref, t_ref, s_ref):
        i = pl.program_id(0)
        u = dinv_ref[...] * (z_ref[0] + z_ref[1] + y_ref[...])
        t = jnp.dot(u, w_ref[...], preferred_element_type=jnp.float32) + b_ref[...]
        t = jnp.maximum(t, 0.0)
        t_ref[...] = t
        rows = i * BLK + lax.broadcasted_iota(jnp.int32, (BLK, 1), 0)
        tm = jnp.where(rows < N, t, 0.0)

        @pl.when(i == 0)
        def _():
            s_ref[...] = jnp.zeros_like(s_ref)

        s_ref[0:1, :] += jnp.sum(tm, axis=0, keepdims=True)

    return pl.pallas_call(
        body,
        grid=(NB,),
        in_specs=[
            pl.BlockSpec((NC, BLK, Fi), lambda i: (0, i, 0)),
            pl.BlockSpec((BLK, Fi), lambda i: (i, 0)),
            pl.BlockSpec((BLK, 1), lambda i: (i, 0)),
            pl.BlockSpec((Fi, Fo), lambda i: (0, 0)),
            pl.BlockSpec((1, Fo), lambda i: (0, 0)),
        ],
        out_specs=[
            pl.BlockSpec((BLK, Fo), lambda i: (i, 0)),
            pl.BlockSpec((8, Fo), lambda i: (0, 0)),
        ],
        out_shape=[
            jax.ShapeDtypeStruct((NPAD, Fo), jnp.float32),
            jax.ShapeDtypeStruct((8, Fo), jnp.float32),
        ],
    )(z_p, y, dinv, W, b2d)


def _tc_bn(t, sums, dinv, g2d, bt2d, scale_by_dinv):
    Fo = t.shape[1]

    def body(t_ref, s_ref, dinv_ref, g_ref, bt_ref, o_ref, v_ref):
        ph = pl.program_id(0)
        i = pl.program_id(1)
        m = s_ref[0:1, :] * (1.0 / N)

        @pl.when(ph == 0)
        def _():
            rows = i * BLK + lax.broadcasted_iota(jnp.int32, (BLK, 1), 0)
            d = jnp.where(rows < N, t_ref[...] - m, 0.0)

            @pl.when(i == 0)
            def _():
                v_ref[...] = jnp.zeros_like(v_ref)

            v_ref[0:1, :] += jnp.sum(d * d, axis=0, keepdims=True)

        @pl.when(ph == 1)
        def _():
            var = v_ref[0:1, :] * (1.0 / N)
            a = g_ref[...] * lax.rsqrt(var + EPS)
            h = a * (t_ref[...] - m) + bt_ref[...]
            if scale_by_dinv:
                h = dinv_ref[...] * h
            o_ref[...] = h

    return pl.pallas_call(
        body,
        grid=(2, NB),
        in_specs=[
            pl.BlockSpec((BLK, Fo), lambda p, i: (i, 0)),
            pl.BlockSpec((8, Fo), lambda p, i: (0, 0)),
            pl.BlockSpec((BLK, 1), lambda p, i: (i, 0)),
            pl.BlockSpec((1, Fo), lambda p, i: (0, 0)),
            pl.BlockSpec((1, Fo), lambda p, i: (0, 0)),
        ],
        out_specs=pl.BlockSpec((BLK, Fo), lambda p, i: (i, 0)),
        out_shape=jax.ShapeDtypeStruct((NPAD, Fo), jnp.float32),
        scratch_shapes=[pltpu.VMEM((8, Fo), jnp.float32)],
    )(t, sums, dinv, g2d, bt2d)


def _tc_layer(z_p, y, dinv, W, b2d, g2d, bt2d, scale_by_dinv):
    Fi, Fo = W.shape

    def body(z_ref, y_ref, dinv_ref, w_ref, b_ref, g_ref, bt_ref, o_ref,
             t_scr, s_scr):
        ph = pl.program_id(0)
        i = pl.program_id(1)
        rows = i * BLK + lax.broadcasted_iota(jnp.int32, (BLK, 1), 0)
        valid = rows < N

        @pl.when(ph == 0)
        def _():
            u = dinv_ref[...] * (z_ref[0] + z_ref[1] + y_ref[...])
            t = jnp.dot(u, w_ref[...], preferred_element_type=jnp.float32)
            t = jnp.maximum(t + b_ref[...], 0.0)
            t_scr[pl.ds(i * BLK, BLK), :] = t

            @pl.when(i == 0)
            def _():
                s_scr[...] = jnp.zeros_like(s_scr)

            s_scr[0:1, :] += jnp.sum(jnp.where(valid, t, 0.0), axis=0,
                                     keepdims=True)

        @pl.when(ph == 1)
        def _():
            m = s_scr[0:1, :] * (1.0 / N)
            d = jnp.where(valid, t_scr[pl.ds(i * BLK, BLK), :] - m, 0.0)
            s_scr[1:2, :] += jnp.sum(d * d, axis=0, keepdims=True)

        @pl.when(ph == 2)
        def _():
            m = s_scr[0:1, :] * (1.0 / N)
            var = s_scr[1:2, :] * (1.0 / N)
            a = g_ref[...] * lax.rsqrt(var + EPS)
            h = a * (t_scr[pl.ds(i * BLK, BLK), :] - m) + bt_ref[...]
            if scale_by_dinv:
                h = dinv_ref[...] * h
            o_ref[...] = h

    return pl.pallas_call(
        body,
        grid=(3, NB),
        in_specs=[
            pl.BlockSpec((NC, BLK, Fi),
                         lambda p, i: (0, jnp.where(p == 0, i, 0), 0)),
            pl.BlockSpec((BLK, Fi),
                         lambda p, i: (jnp.where(p == 0, i, 0), 0)),
            pl.BlockSpec((BLK, 1), lambda p, i: (i, 0)),
            pl.BlockSpec((Fi, Fo), lambda p, i: (0, 0)),
            pl.BlockSpec((1, Fo), lambda p, i: (0, 0)),
            pl.BlockSpec((1, Fo), lambda p, i: (0, 0)),
            pl.BlockSpec((1, Fo), lambda p, i: (0, 0)),
        ],
        out_specs=pl.BlockSpec((BLK, Fo),
                               lambda p, i: (jnp.where(p == 2, i, 0), 0)),
        out_shape=jax.ShapeDtypeStruct((NPAD, Fo), jnp.float32),
        scratch_shapes=[
            pltpu.VMEM((NPAD, Fo), jnp.float32),
            pltpu.VMEM((8, Fo), jnp.float32),
        ],
    )(z_p, y, dinv, W, b2d, g2d, bt2d)


def _tc_head(pool_p, Wl, bl2d):
    def body(p_ref, w_ref, b_ref, o_ref):
        p = jnp.maximum(p_ref[0], p_ref[1])[:G]
        v = jnp.dot(p, w_ref[...], preferred_element_type=jnp.float32) + b_ref[...]
        mx = jnp.max(v, axis=1, keepdims=True)
        e = jnp.exp(v - mx)
        o_ref[...] = (v - mx) - jnp.log(jnp.sum(e, axis=1, keepdims=True))

    return pl.pallas_call(
        body,
        out_shape=jax.ShapeDtypeStruct((G, 3), jnp.float32),
    )(pool_p, Wl, bl2d)


def kernel(x, edge_index, batch_index, W1, b1, g1, bt1, W2, b2, g2, bt2,
           W3, b3, g3, bt3, Wl, bl):
    i32 = jnp.int32
    src_pad = jnp.concatenate([edge_index[0].astype(i32),
                               jnp.zeros((EPAD - E,), i32)]).reshape(EPAD // CH, CH)
    dst_pad = jnp.concatenate([edge_index[1].astype(i32),
                               jnp.full((EPAD - E,), N, i32)]).reshape(EPAD // CH, CH)
    batch_pad = jnp.concatenate([batch_index.astype(i32),
                                 jnp.full((NPAD - N,), G, i32)])
    x_pad = jnp.zeros((NPAD, 8), jnp.float32).at[:N, :2].set(x)

    ones_rows = jnp.ones((CH, 8), jnp.float32)
    zrows1 = jnp.zeros((NPAD, 8), jnp.float32)
    neginf_rows = jnp.full((GPAD, 128), -jnp.inf, jnp.float32)

    deg_p = _sc_degree(dst_pad, ones_rows, zrows1)
    dinv, y = _tc_prep(deg_p, x_pad)

    W1p = jnp.zeros((8, 8), jnp.float32).at[:2].set(W1)
    layers = [(W1p, b1, g1, bt1), (W2, b2, g2, bt2), (W3, b3, g3, bt3)]
    for li, (W, b, g, bt) in enumerate(layers):
        Fi = W.shape[0]
        z_p = _sc_agg(Fi)(y, src_pad, dst_pad, jnp.zeros((NPAD, Fi), jnp.float32))
        y = _tc_layer(z_p, y, dinv, W, b.reshape(1, -1), g.reshape(1, -1),
                      bt.reshape(1, -1), scale_by_dinv=(li < 2))

    pool_p = _sc_pool(y, batch_pad, neginf_rows)
    return _tc_head(pool_p, Wl, bl.reshape(1, -1))


# speedup vs baseline: 1.1654x; 1.0358x over previous
import functools

import jax
import jax.numpy as jnp
from jax import lax
from jax.experimental import pallas as pl
from jax.experimental.pallas import tpu as pltpu
from jax.experimental.pallas import tpu_sc as plsc

N = 50000
E = 800000
G = 128

NC = 2
NS = 16
NW = NC * NS

NPAD = 50048
NB = 16
BLK = NPAD // NB

CH = 128
EPT = 25088
EPAD = NW * EPT
NCHUNK = EPT // CH

GPAD = 144
EPS = 1e-5

_mesh = lambda: plsc.VectorSubcoreMesh(core_axis_name="c", subcore_axis_name="s")
_sc_params = lambda: pltpu.CompilerParams(use_tc_tiling_on_sc=False)


def _sc_degree(dst_pad, ones_rows, zrows):
    @functools.partial(
        pl.kernel,
        out_type=jax.ShapeDtypeStruct((NC, NPAD, 8), jnp.float32),
        mesh=_mesh(),
        compiler_params=_sc_params(),
        scratch_types=[
            pltpu.VMEM((NCHUNK, CH), jnp.int32),
            pltpu.VMEM((CH, 8), jnp.float32),
            pltpu.VMEM_SHARED((NPAD, 8), jnp.float32),
        ],
    )
    def deg_kernel(dst_hbm, ones_hbm, zeros_hbm, out_hbm, didx, ones_v, acc):
        c = lax.axis_index("c")
        s = lax.axis_index("s")
        wid = c * NS + s
        rows_t = NPAD // NS
        r0 = s * rows_t
        pltpu.sync_copy(ones_hbm, ones_v)
        pltpu.sync_copy(zeros_hbm.at[pl.ds(r0, rows_t)], acc.at[pl.ds(r0, rows_t)])
        pltpu.sync_copy(dst_hbm.at[pl.ds(wid * NCHUNK, NCHUNK)], didx)
        plsc.subcore_barrier()

        def body(k, carry):
            pltpu.sync_copy(ones_v, acc.at[didx.at[k]], add=True)
            return carry

        lax.fori_loop(0, NCHUNK, body, 0)
        plsc.subcore_barrier()
        pltpu.sync_copy(acc.at[pl.ds(r0, rows_t)], out_hbm.at[c, pl.ds(r0, rows_t)])

    return deg_kernel(dst_pad, ones_rows, zrows)


def _sc_agg(F):
    cpb = NCHUNK if F <= 8 else NCHUNK // 4
    nblk = NCHUNK // cpb

    @functools.partial(
        pl.kernel,
        out_type=jax.ShapeDtypeStruct((NC, NPAD, F), jnp.float32),
        mesh=_mesh(),
        compiler_params=_sc_params(),
        scratch_types=[
            pltpu.VMEM((cpb, CH), jnp.int32),
            pltpu.VMEM((cpb, CH), jnp.int32),
            pltpu.VMEM((4, CH, F), jnp.float32),
            pltpu.VMEM_SHARED((NPAD, F), jnp.float32),
            [pltpu.SemaphoreType.DMA] * 4,
            [pltpu.SemaphoreType.DMA] * 4,
        ],
    )
    def agg_kernel(y_hbm, src_hbm, dst_hbm, zeros_hbm, out_hbm,
                   sidx, didx, rows, acc, gsems, ssems):
        c = lax.axis_index("c")
        s = lax.axis_index("s")
        wid = c * NS + s
        rows_t = NPAD // NS
        r0 = s * rows_t
        pltpu.sync_copy(zeros_hbm.at[pl.ds(r0, rows_t)], acc.at[pl.ds(r0, rows_t)])
        plsc.subcore_barrier()

        for blk in range(nblk):
            cb = wid * NCHUNK + blk * cpb
            pltpu.sync_copy(src_hbm.at[pl.ds(cb, cpb)], sidx)
            pltpu.sync_copy(dst_hbm.at[pl.ds(cb, cpb)], didx)
            for b in range(2):
                pltpu.async_copy(y_hbm.at[sidx.at[b]], rows.at[b], gsems[b])

            def step(k, b, guard):
                b2 = (b + 2) % 4
                pltpu.make_async_copy(y_hbm.at[sidx.at[k]], rows.at[b],
                                      gsems[b]).wait()
                pltpu.async_copy(rows.at[b], acc.at[didx.at[k]], ssems[b],
                                 add=True)
                if guard:
                    @pl.when(k >= 2)
                    def _():
                        pltpu.make_async_copy(rows.at[b2], acc.at[didx.at[0]],
                                              ssems[b2]).wait()
                    @pl.when(k + 2 < cpb)
                    def _():
                        pltpu.async_copy(y_hbm.at[sidx.at[k + 2]], rows.at[b2],
                                         gsems[b2])

            def body(kk, carry):
                for j in range(4):
                    step(kk * 4 + j, j, True)
                return carry

            lax.fori_loop(0, cpb // 4, body, 0)
            for k in range((cpb // 4) * 4, cpb):
                b, b2 = k % 4, (k + 2) % 4
                pltpu.make_async_copy(y_hbm.at[sidx.at[k]], rows.at[b],
                                      gsems[b]).wait()
                pltpu.async_copy(rows.at[b], acc.at[didx.at[k]], ssems[b],
                                 add=True)
                pltpu.make_async_copy(rows.at[b2], acc.at[didx.at[0]],
                                      ssems[b2]).wait()
                if k + 2 < cpb:
                    pltpu.async_copy(y_hbm.at[sidx.at[k + 2]], rows.at[b2],
                                     gsems[b2])
            for k in (cpb - 2, cpb - 1):
                pltpu.make_async_copy(rows.at[k % 4], acc.at[didx.at[0]],
                                      ssems[k % 4]).wait()

        plsc.subcore_barrier()
        pltpu.sync_copy(acc.at[pl.ds(r0, rows_t)], out_hbm.at[c, pl.ds(r0, rows_t)])

    return agg_kernel


def _sc_pool(h_pad, batch_pad, neginf_rows):
    nchunks_total = NPAD // CH

    @functools.partial(
        pl.kernel,
        out_type=jax.ShapeDtypeStruct((NC, GPAD, 128), jnp.float32),
        mesh=_mesh(),
        compiler_params=_sc_params(),
        scratch_types=[
            pltpu.VMEM((CH,), jnp.int32),
            pltpu.VMEM((CH, 128), jnp.float32),
            pltpu.VMEM((GPAD, 128), jnp.float32),
            pltpu.VMEM((2, 9, 128), jnp.float32),
            pltpu.VMEM_SHARED((NS, GPAD, 128), jnp.float32),
        ],
    )
    def pool_kernel(h_hbm, b_hbm, neg_hbm, out_hbm, bidx, hrows, acc, rbuf, shared):
        c = lax.axis_index("c")
        s = lax.axis_index("s")
        wid = c * NS + s
        pltpu.sync_copy(neg_hbm, acc)
        nch = (nchunks_total - wid + NW - 1) // NW

        neg = jnp.full((16,), -jnp.inf, jnp.float32)

        def flush(prev, run):
            @pl.when(prev >= 0)
            def _():
                for j in range(8):
                    a = acc[prev, pl.ds(j * 16, 16)]
                    acc[prev, pl.ds(j * 16, 16)] = jnp.maximum(a, run[j])

        def chunk_body(k, carry):
            base = (wid + k * NW) * CH
            pltpu.sync_copy(b_hbm.at[pl.ds(base, CH)], bidx)
            pltpu.sync_copy(h_hbm.at[pl.ds(base, CH)], hrows)

            def group_body(q, carry2):
                prev = carry2[0]
                run = list(carry2[1:])
                bvec = bidx[pl.ds(q * 16, 16)]
                for r in range(16):
                    bi = bvec[r]
                    rr = q * 16 + r
                    new_seg = bi != prev
                    flush_now = jnp.logical_and(new_seg, prev >= 0)

                    @pl.when(flush_now)
                    def _(run=run, prev=prev):
                        for j in range(8):
                            a = acc[prev, pl.ds(j * 16, 16)]
                            acc[prev, pl.ds(j * 16, 16)] = jnp.maximum(a, run[j])

                    for j in range(8):
                        v = hrows[rr, pl.ds(j * 16, 16)]
                        run[j] = jnp.where(new_seg, v, jnp.maximum(run[j], v))
                    prev = bi
                return (prev, *run)

            return lax.fori_loop(0, CH // 16, group_body, carry)

        fin = lax.fori_loop(0, nch, chunk_body,
                            (jnp.int32(-1),) + tuple(neg for _ in range(8)))
        flush(fin[0], list(fin[1:]))
        pltpu.sync_copy(acc, shared.at[s])
        plsc.subcore_barrier()
        g0 = s * 9
        pltpu.sync_copy(shared.at[0, pl.ds(g0, 9)], rbuf.at[0])

        def red_body(i, carry):
            pltpu.sync_copy(shared.at[i, pl.ds(g0, 9)], rbuf.at[1])
            for r in range(9):
                for j in range(8):
                    a = rbuf[0, r, pl.ds(j * 16, 16)]
                    v = rbuf[1, r, pl.ds(j * 16, 16)]
                    rbuf[0, r, pl.ds(j * 16, 16)] = jnp.maximum(a, v)
            return carry

        lax.fori_loop(1, NS, red_body, 0)
        pltpu.sync_copy(rbuf.at[0], out_hbm.at[c, pl.ds(g0, 9)])

    return pool_kernel(h_pad, batch_pad, neginf_rows)


def _tc_prep(deg_p, x_pad):
    def body(dp_ref, x_ref, dinv_ref, y1_ref):
        d = 1.0 + dp_ref[0, :, 0:1] + dp_ref[1, :, 0:1]
        dinv = lax.rsqrt(d)
        dinv_ref[...] = dinv
        y1_ref[...] = dinv * x_ref[...]

    return pl.pallas_call(
        body,
        grid=(NB,),
        in_specs=[
            pl.BlockSpec((NC, BLK, 8), lambda i: (0, i, 0)),
            pl.BlockSpec((BLK, 8), lambda i: (i, 0)),
        ],
        out_specs=[
            pl.BlockSpec((BLK, 1), lambda i: (i, 0)),
            pl.BlockSpec((BLK, 8), lambda i: (i, 0)),
        ],
        out_shape=[
            jax.ShapeDtypeStruct((NPAD, 1), jnp.float32),
            jax.ShapeDtypeStruct((NPAD, 8), jnp.float32),
        ],
    )(deg_p, x_pad)


def _tc_conv(z_p, y, dinv, W, b2d):
    Fi, Fo = W.shape

    def body(z_ref, y_ref, dinv_ref, w_ref, b_ref, t_ref, s_ref):
        i = pl.program_id(0)
        u = dinv_ref[...] * (z_ref[0] + z_ref[1] + y_ref[...])
        t = jnp.dot(u, w_ref[...], preferred_element_type=jnp.float32) + b_ref[...]
        t = jnp.maximum(t, 0.0)
        t_ref[...] = t
        rows = i * BLK + lax.broadcasted_iota(jnp.int32, (BLK, 1), 0)
        tm = jnp.where(rows < N, t, 0.0)

        @pl.when(i == 0)
        def _():
            s_ref[...] = jnp.zeros_like(s_ref)

        s_ref[0:1, :] += jnp.sum(tm, axis=0, keepdims=True)

    return pl.pallas_call(
        body,
        grid=(NB,),
        in_specs=[
            pl.BlockSpec((NC, BLK, Fi), lambda i: (0, i, 0)),
            pl.BlockSpec((BLK, Fi), lambda i: (i, 0)),
            pl.BlockSpec((BLK, 1), lambda i: (i, 0)),
            pl.BlockSpec((Fi, Fo), lambda i: (0, 0)),
            pl.BlockSpec((1, Fo), lambda i: (0, 0)),
        ],
        out_specs=[
            pl.BlockSpec((BLK, Fo), lambda i: (i, 0)),
            pl.BlockSpec((8, Fo), lambda i: (0, 0)),
        ],
        out_shape=[
            jax.ShapeDtypeStruct((NPAD, Fo), jnp.float32),
            jax.ShapeDtypeStruct((8, Fo), jnp.float32),
        ],
    )(z_p, y, dinv, W, b2d)


def _tc_bn(t, sums, dinv, g2d, bt2d, scale_by_dinv):
    Fo = t.shape[1]

    def body(t_ref, s_ref, dinv_ref, g_ref, bt_ref, o_ref, v_ref):
        ph = pl.program_id(0)
        i = pl.program_id(1)
        m = s_ref[0:1, :] * (1.0 / N)

        @pl.when(ph == 0)
        def _():
            rows = i * BLK + lax.broadcasted_iota(jnp.int32, (BLK, 1), 0)
            d = jnp.where(rows < N, t_ref[...] - m, 0.0)

            @pl.when(i == 0)
            def _():
                v_ref[...] = jnp.zeros_like(v_ref)

            v_ref[0:1, :] += jnp.sum(d * d, axis=0, keepdims=True)

        @pl.when(ph == 1)
        def _():
            var = v_ref[0:1, :] * (1.0 / N)
            a = g_ref[...] * lax.rsqrt(var + EPS)
            h = a * (t_ref[...] - m) + bt_ref[...]
            if scale_by_dinv:
                h = dinv_ref[...] * h
            o_ref[...] = h

    return pl.pallas_call(
        body,
        grid=(2, NB),
        in_specs=[
            pl.BlockSpec((BLK, Fo), lambda p, i: (i, 0)),
            pl.BlockSpec((8, Fo), lambda p, i: (0, 0)),
            pl.BlockSpec((BLK, 1), lambda p, i: (i, 0)),
            pl.BlockSpec((1, Fo), lambda p, i: (0, 0)),
            pl.BlockSpec((1, Fo), lambda p, i: (0, 0)),
        ],
        out_specs=pl.BlockSpec((BLK, Fo), lambda p, i: (i, 0)),
        out_shape=jax.ShapeDtypeStruct((NPAD, Fo), jnp.float32),
        scratch_shapes=[pltpu.VMEM((8, Fo), jnp.float32)],
    )(t, sums, dinv, g2d, bt2d)


def _tc_layer(z_p, y, dinv, W, b2d, g2d, bt2d, scale_by_dinv):
    Fi, Fo = W.shape

    def body(z_ref, y_ref, dinv_ref, w_ref, b_ref, g_ref, bt_ref, o_ref,
             t_scr, s_scr):
        ph = pl.program_id(0)
        i = pl.program_id(1)
        rows = i * BLK + lax.broadcasted_iota(jnp.int32, (BLK, 1), 0)
        valid = rows < N

        @pl.when(ph == 0)
        def _():
            u = dinv_ref[...] * (z_ref[0] + z_ref[1] + y_ref[...])
            t = jnp.dot(u, w_ref[...], preferred_element_type=jnp.float32)
            t = jnp.maximum(t + b_ref[...], 0.0)
            t_scr[pl.ds(i * BLK, BLK), :] = t

            @pl.when(i == 0)
            def _():
                s_scr[...] = jnp.zeros_like(s_scr)

            s_scr[0:1, :] += jnp.sum(jnp.where(valid, t, 0.0), axis=0,
                                     keepdims=True)

        @pl.when(ph == 1)
        def _():
            m = s_scr[0:1, :] * (1.0 / N)
            d = jnp.where(valid, t_scr[pl.ds(i * BLK, BLK), :] - m, 0.0)
            s_scr[1:2, :] += jnp.sum(d * d, axis=0, keepdims=True)

        @pl.when(ph == 2)
        def _():
            m = s_scr[0:1, :] * (1.0 / N)
            var = s_scr[1:2, :] * (1.0 / N)
            a = g_ref[...] * lax.rsqrt(var + EPS)
            h = a * (t_scr[pl.ds(i * BLK, BLK), :] - m) + bt_ref[...]
            if scale_by_dinv:
                h = dinv_ref[...] * h
            o_ref[...] = h

    return pl.pallas_call(
        body,
        grid=(3, NB),
        in_specs=[
            pl.BlockSpec((NC, BLK, Fi),
                         lambda p, i: (0, jnp.where(p == 0, i, 0), 0)),
            pl.BlockSpec((BLK, Fi),
                         lambda p, i: (jnp.where(p == 0, i, 0), 0)),
            pl.BlockSpec((BLK, 1), lambda p, i: (i, 0)),
            pl.BlockSpec((Fi, Fo), lambda p, i: (0, 0)),
            pl.BlockSpec((1, Fo), lambda p, i: (0, 0)),
            pl.BlockSpec((1, Fo), lambda p, i: (0, 0)),
            pl.BlockSpec((1, Fo), lambda p, i: (0, 0)),
        ],
        out_specs=pl.BlockSpec((BLK, Fo),
                               lambda p, i: (jnp.where(p == 2, i, 0), 0)),
        out_shape=jax.ShapeDtypeStruct((NPAD, Fo), jnp.float32),
        scratch_shapes=[
            pltpu.VMEM((NPAD, Fo), jnp.float32),
            pltpu.VMEM((8, Fo), jnp.float32),
        ],
    )(z_p, y, dinv, W, b2d, g2d, bt2d)


def _tc_head(pool_p, Wl, bl2d):
    def body(p_ref, w_ref, b_ref, o_ref):
        p = jnp.maximum(p_ref[0], p_ref[1])[:G]
        v = jnp.dot(p, w_ref[...], preferred_element_type=jnp.float32) + b_ref[...]
        mx = jnp.max(v, axis=1, keepdims=True)
        e = jnp.exp(v - mx)
        o_ref[...] = (v - mx) - jnp.log(jnp.sum(e, axis=1, keepdims=True))

    return pl.pallas_call(
        body,
        out_shape=jax.ShapeDtypeStruct((G, 3), jnp.float32),
    )(pool_p, Wl, bl2d)


def kernel(x, edge_index, batch_index, W1, b1, g1, bt1, W2, b2, g2, bt2,
           W3, b3, g3, bt3, Wl, bl):
    i32 = jnp.int32
    src_pad = jnp.concatenate([edge_index[0].astype(i32),
                               jnp.zeros((EPAD - E,), i32)]).reshape(EPAD // CH, CH)
    dst_pad = jnp.concatenate([edge_index[1].astype(i32),
                               jnp.full((EPAD - E,), N, i32)]).reshape(EPAD // CH, CH)
    batch_pad = jnp.concatenate([batch_index.astype(i32),
                                 jnp.full((NPAD - N,), G, i32)])
    x_pad = jnp.zeros((NPAD, 8), jnp.float32).at[:N, :2].set(x)

    ones_rows = jnp.ones((CH, 8), jnp.float32)
    zrows1 = jnp.zeros((NPAD, 8), jnp.float32)
    neginf_rows = jnp.full((GPAD, 128), -jnp.inf, jnp.float32)

    deg_p = _sc_degree(dst_pad, ones_rows, zrows1)
    dinv, y = _tc_prep(deg_p, x_pad)

    W1p = jnp.zeros((8, 8), jnp.float32).at[:2].set(W1)
    layers = [(W1p, b1, g1, bt1), (W2, b2, g2, bt2), (W3, b3, g3, bt3)]
    for li, (W, b, g, bt) in enumerate(layers):
        Fi = W.shape[0]
        z_p = _sc_agg(Fi)(y, src_pad, dst_pad, jnp.zeros((NPAD, Fi), jnp.float32))
        y = _tc_layer(z_p, y, dinv, W, b.reshape(1, -1), g.reshape(1, -1),
                      bt.reshape(1, -1), scale_by_dinv=(li < 2))

    pool_p = _sc_pool(y, batch_pad, neginf_rows)
    return _tc_head(pool_p, Wl, bl.reshape(1, -1))
